# Initial kernel scaffold; baseline (speedup 1.0000x reference)
import jax
import jax.numpy as jnp
from jax.experimental import pallas as pl


def kernel(coord, rpe_table):
    raise NotImplementedError("write your pallas kernel here")


# speedup vs baseline: 8.2609x; 8.2609x over previous
import functools

import jax
import jax.numpy as jnp
from jax import lax
from jax.experimental import pallas as pl
from jax.experimental.pallas import tpu as pltpu
from jax.experimental.pallas import tpu_sc as plsc

PATCH = 1024
HEADS = 16
POS_BND = int((4 * PATCH) ** (1 / 3) * 2)
RPE_NUM = 2 * POS_BND + 1

B = 8
S = 512
N = B * S * S
NC, NS, L = 2, 16, 16
NW = NC * NS
PER_W = N // NW
BP = 1024
NBLK = PER_W // BP
NGRP = BP // L
PER_B = S * S
Q_PER_B = PER_B // PER_W

_T01_ROWS = RPE_NUM * RPE_NUM
_C01 = POS_BND * RPE_NUM * HEADS + POS_BND * HEADS
_C2 = POS_BND * HEADS


def _rpe_body(t01_hbm, t2_hbm, coord_hbm, out_hbm, t01_v, t2_v, coord_v, out_v):
    wid = lax.axis_index("c") * NS + lax.axis_index("s")
    b = wid // Q_PER_B
    q = wid % Q_PER_B

    pltpu.sync_copy(t01_hbm, t01_v)
    pltpu.sync_copy(t2_hbm, t2_v)

    iota = lax.iota(jnp.int32, L)
    iota3 = iota * 3

    def block_body(blk, _):
        base_pos = wid * PER_W + blk * BP
        pltpu.sync_copy(coord_hbm.at[pl.ds(base_pos * 3, BP * 3)], coord_v)

        def group_body(g, _):
            cbase = iota3 + g * (3 * L)
            c0 = plsc.load_gather(coord_v, [cbase])
            c1 = plsc.load_gather(coord_v, [cbase + 1])
            c2 = plsc.load_gather(coord_v, [cbase + 2])
            c0 = jnp.minimum(jnp.maximum(c0, -POS_BND), POS_BND)
            c1 = jnp.minimum(jnp.maximum(c1, -POS_BND), POS_BND)
            c2 = jnp.minimum(jnp.maximum(c2, -POS_BND), POS_BND)
            flat01 = c0 * (RPE_NUM * HEADS) + c1 * HEADS + _C01
            flat2 = c2 * HEADS + _C2
            col = g * L
            for h in range(HEADS):
                v01 = plsc.load_gather(t01_v, [flat01 + h])
                v2 = plsc.load_gather(t2_v, [flat2 + h])
                out_v[h, pl.ds(col, L)] = v01 + v2
            return 0

        lax.fori_loop(0, NGRP, group_body, 0)
        pltpu.sync_copy(out_v, out_hbm.at[b, :, pl.ds(q * PER_W + blk * BP, BP)])
        return 0

    lax.fori_loop(0, NBLK, block_body, 0)


_rpe_call = functools.partial(
    pl.kernel,
    out_type=jax.ShapeDtypeStruct((B, HEADS, PER_B), jnp.float32),
    mesh=plsc.VectorSubcoreMesh(
        core_axis_name="c", subcore_axis_name="s", num_cores=NC, num_subcores=NS
    ),
    scratch_types=[
        pltpu.VMEM((_T01_ROWS * HEADS,), jnp.float32),
        pltpu.VMEM((RPE_NUM * HEADS,), jnp.float32),
        pltpu.VMEM((BP * 3,), jnp.int32),
        pltpu.VMEM((HEADS, BP), jnp.float32),
    ],
    compiler_params=pltpu.CompilerParams(needs_layout_passes=False),
)(_rpe_body)


def kernel(coord, rpe_table):
    t0 = rpe_table[0:RPE_NUM]
    t1 = rpe_table[RPE_NUM : 2 * RPE_NUM]
    t2 = rpe_table[2 * RPE_NUM : 3 * RPE_NUM]
    t01 = (t0[:, None, :] + t1[None, :, :]).reshape(-1)
    out = _rpe_call(t01, t2.reshape(-1), coord.reshape(-1))
    return out.reshape(B, HEADS, S, S)


# speedup vs baseline: 10.2171x; 1.2368x over previous
import functools

import jax
import jax.numpy as jnp
from jax import lax
from jax.experimental import pallas as pl
from jax.experimental.pallas import tpu as pltpu
from jax.experimental.pallas import tpu_sc as plsc

PATCH = 1024
HEADS = 16
HPAIRS = HEADS // 2
POS_BND = int((4 * PATCH) ** (1 / 3) * 2)
RPE_NUM = 2 * POS_BND + 1

B = 8
S = 512
N = B * S * S
NC, NS, L = 2, 16, 16
NW = NC * NS
PER_W = N // NW
BP = 4096
BPR = BP // S
NBLK = PER_W // BP
NGRP = BP // L
GPR = S // L
Q_PER_B = (S * S) // PER_W
ROWS_W = PER_W // S

_T01_ROWS = RPE_NUM * RPE_NUM
_C01 = (POS_BND * RPE_NUM + POS_BND) * HPAIRS
_C2 = POS_BND * HPAIRS


def _rpe_body(t01_hbm, t2_hbm, coord_hbm, out_hbm, t01_v, t2_v, coord_v, out_v):
    wid = lax.axis_index("c") * NS + lax.axis_index("s")
    b = wid // Q_PER_B
    q = wid % Q_PER_B

    pltpu.sync_copy(t01_hbm, t01_v)
    pltpu.sync_copy(t2_hbm, t2_v)

    iota = lax.iota(jnp.int32, L)
    iota3 = iota * 3

    def block_body(blk, _):
        base_pos = wid * PER_W + blk * BP
        pltpu.sync_copy(coord_hbm.at[pl.ds(base_pos * 3, BP * 3)], coord_v)

        def group_body(g, _):
            cbase = iota3 + g * (3 * L)
            c0 = plsc.load_gather(coord_v, [cbase])
            c1 = plsc.load_gather(coord_v, [cbase + 1])
            c2 = plsc.load_gather(coord_v, [cbase + 2])
            c0 = jnp.minimum(jnp.maximum(c0, -POS_BND), POS_BND)
            c1 = jnp.minimum(jnp.maximum(c1, -POS_BND), POS_BND)
            c2 = jnp.minimum(jnp.maximum(c2, -POS_BND), POS_BND)
            p01 = c0 * (RPE_NUM * HPAIRS) + c1 * HPAIRS + _C01
            p2 = c2 * HPAIRS + _C2
            r = g // GPR
            col = (g % GPR) * L
            for k in range(HPAIRS):
                w01 = plsc.load_gather(t01_v, [p01 + k])
                w2 = plsc.load_gather(t2_v, [p2 + k])
                s = plsc.bitcast(w01, jnp.bfloat16) + plsc.bitcast(w2, jnp.bfloat16)
                lo, hi = plsc.unpack(
                    s, format=plsc.PackFormat.INTERLEAVED,
                    preferred_element_type=jnp.float32,
                )
                out_v[2 * k, r, pl.ds(col, L)] = lo
                out_v[2 * k + 1, r, pl.ds(col, L)] = hi
            return 0

        lax.fori_loop(0, NGRP, group_body, 0)
        row0 = q * ROWS_W + blk * BPR
        pltpu.sync_copy(out_v, out_hbm.at[b, :, pl.ds(row0, BPR), :])
        return 0

    lax.fori_loop(0, NBLK, block_body, 0)


_rpe_call = functools.partial(
    pl.kernel,
    out_type=jax.ShapeDtypeStruct((B, HEADS, S, S), jnp.float32),
    mesh=plsc.VectorSubcoreMesh(
        core_axis_name="c", subcore_axis_name="s", num_cores=NC, num_subcores=NS
    ),
    scratch_types=[
        pltpu.VMEM((_T01_ROWS * HPAIRS,), jnp.int32),
        pltpu.VMEM((RPE_NUM * HPAIRS,), jnp.int32),
        pltpu.VMEM((BP * 3,), jnp.int32),
        pltpu.VMEM((HEADS, BPR, S), jnp.float32),
    ],
    compiler_params=pltpu.CompilerParams(
        needs_layout_passes=False, use_tc_tiling_on_sc=True
    ),
)(_rpe_body)


def _pack_pairs(t):
    tb = t.astype(jnp.bfloat16).reshape(t.shape[0], HPAIRS, 2)
    return lax.bitcast_convert_type(tb, jnp.int32).reshape(-1)


def kernel(coord, rpe_table):
    t0 = rpe_table[0:RPE_NUM]
    t1 = rpe_table[RPE_NUM : 2 * RPE_NUM]
    t2 = rpe_table[2 * RPE_NUM : 3 * RPE_NUM]
    t01 = t0[:, None, :] + t1[None, :, :]
    return _rpe_call(
        _pack_pairs(t01.reshape(_T01_ROWS, HEADS)),
        _pack_pairs(t2),
        coord.reshape(-1),
    )


# speedup vs baseline: 67.1359x; 6.5709x over previous
import functools

import jax
import jax.numpy as jnp
from jax import lax
from jax.experimental import pallas as pl
from jax.experimental.pallas import tpu as pltpu
from jax.experimental.pallas import tpu_sc as plsc

PATCH = 1024
HEADS = 16
HPAIRS = HEADS // 2
POS_BND = int((4 * PATCH) ** (1 / 3) * 2)
RPE_NUM = 2 * POS_BND + 1

B = 8
S = 512
N = B * S * S
NC, NS, L = 2, 16, 16
NW = NC * NS
PER_W = N // NW
BP = 4096
BPR = BP // S
NBLK = PER_W // BP
NGRP = BP // L
GPR = S // L
Q_PER_B = (S * S) // PER_W
ROWS_W = PER_W // S

_T01_ROWS = RPE_NUM * RPE_NUM
_C01 = (POS_BND * RPE_NUM + POS_BND) * HPAIRS
_C2 = POS_BND * HPAIRS


def _rpe_body(t01_hbm, t2_hbm, coord_hbm, out_hbm, t01_v, t2_v, coord_v, out_v):
    wid = lax.axis_index("c") * NS + lax.axis_index("s")
    b = wid // Q_PER_B
    q = wid % Q_PER_B

    pltpu.sync_copy(t01_hbm, t01_v)
    pltpu.sync_copy(t2_hbm, t2_v)

    def block_body(blk, _):
        brow0 = q * ROWS_W + blk * BPR
        pltpu.sync_copy(coord_hbm.at[b, :, pl.ds(brow0, BPR), :], coord_v)

        def group_body(g, _):
            r = g // GPR
            col = (g % GPR) * L
            c0 = coord_v[0, r, pl.ds(col, L)]
            c1 = coord_v[1, r, pl.ds(col, L)]
            c2 = coord_v[2, r, pl.ds(col, L)]
            c0 = jnp.minimum(jnp.maximum(c0, -POS_BND), POS_BND)
            c1 = jnp.minimum(jnp.maximum(c1, -POS_BND), POS_BND)
            c2 = jnp.minimum(jnp.maximum(c2, -POS_BND), POS_BND)
            p01 = c0 * (RPE_NUM * HPAIRS) + c1 * HPAIRS + _C01
            p2 = c2 * HPAIRS + _C2
            for k in range(HPAIRS):
                w01 = plsc.load_gather(t01_v, [p01 + k])
                w2 = plsc.load_gather(t2_v, [p2 + k])
                s = plsc.bitcast(w01, jnp.bfloat16) + plsc.bitcast(w2, jnp.bfloat16)
                lo, hi = plsc.unpack(
                    s, format=plsc.PackFormat.INTERLEAVED,
                    preferred_element_type=jnp.float32,
                )
                out_v[2 * k, r, pl.ds(col, L)] = lo
                out_v[2 * k + 1, r, pl.ds(col, L)] = hi
            return 0

        lax.fori_loop(0, NGRP, group_body, 0)
        pltpu.sync_copy(out_v, out_hbm.at[b, :, pl.ds(brow0, BPR), :])
        return 0

    lax.fori_loop(0, NBLK, block_body, 0)


_rpe_call = functools.partial(
    pl.kernel,
    out_type=jax.ShapeDtypeStruct((B, HEADS, S, S), jnp.float32),
    mesh=plsc.VectorSubcoreMesh(
        core_axis_name="c", subcore_axis_name="s", num_cores=NC, num_subcores=NS
    ),
    scratch_types=[
        pltpu.VMEM((_T01_ROWS * HPAIRS,), jnp.int32),
        pltpu.VMEM((RPE_NUM * HPAIRS,), jnp.int32),
        pltpu.VMEM((3, BPR, S), jnp.int32),
        pltpu.VMEM((HEADS, BPR, S), jnp.float32),
    ],
    compiler_params=pltpu.CompilerParams(
        needs_layout_passes=False, use_tc_tiling_on_sc=True
    ),
)(_rpe_body)


def _pack_pairs(t):
    tb = t.astype(jnp.bfloat16).reshape(t.shape[0], HPAIRS, 2)
    return lax.bitcast_convert_type(tb, jnp.int32).reshape(-1)


def kernel(coord, rpe_table):
    t0 = rpe_table[0:RPE_NUM]
    t1 = rpe_table[RPE_NUM : 2 * RPE_NUM]
    t2 = rpe_table[2 * RPE_NUM : 3 * RPE_NUM]
    t01 = t0[:, None, :] + t1[None, :, :]
    return _rpe_call(
        _pack_pairs(t01.reshape(_T01_ROWS, HEADS)),
        _pack_pairs(t2),
        jnp.transpose(coord, (0, 3, 1, 2)),
    )


# speedup vs baseline: 106.4762x; 1.5860x over previous
import functools

import jax
import jax.numpy as jnp
from jax import lax
from jax.experimental import pallas as pl
from jax.experimental.pallas import tpu as pltpu
from jax.experimental.pallas import tpu_sc as plsc

PATCH = 1024
HEADS = 16
HPAIRS = HEADS // 2
POS_BND = int((4 * PATCH) ** (1 / 3) * 2)
RPE_NUM = 2 * POS_BND + 1

B = 8
S = 512
N = B * S * S
NC, NS, L = 2, 16, 16
NW = NC * NS
PER_W = N // NW
BP = 4096
BPR = BP // S
NBLK = PER_W // BP
NGRP = BP // L
GPR = S // L
Q_PER_B = (S * S) // PER_W
ROWS_W = PER_W // S

_T01_ROWS = RPE_NUM * RPE_NUM
_C01 = (POS_BND * RPE_NUM + POS_BND) * HPAIRS
_C2 = POS_BND * HPAIRS


def _rpe_body(t01_hbm, t2_hbm, coord_hbm, out_hbm, t01_v, t2_v, coord_v, out_v):
    wid = lax.axis_index("c") * NS + lax.axis_index("s")
    b = wid // Q_PER_B
    q = wid % Q_PER_B

    pltpu.sync_copy(t01_hbm, t01_v)
    pltpu.sync_copy(t2_hbm, t2_v)

    def block_body(blk, _):
        brow0 = q * ROWS_W + blk * BPR
        pltpu.sync_copy(coord_hbm.at[b, :, pl.ds(brow0, BPR), :], coord_v)

        @plsc.parallel_loop(0, NGRP, unroll=4)
        def group_body(g):
            r = g // GPR
            col = (g % GPR) * L
            c0 = coord_v[0, r, pl.ds(col, L)]
            c1 = coord_v[1, r, pl.ds(col, L)]
            c2 = coord_v[2, r, pl.ds(col, L)]
            c0 = jnp.minimum(jnp.maximum(c0, -POS_BND), POS_BND)
            c1 = jnp.minimum(jnp.maximum(c1, -POS_BND), POS_BND)
            c2 = jnp.minimum(jnp.maximum(c2, -POS_BND), POS_BND)
            p01 = c0 * (RPE_NUM * HPAIRS) + c1 * HPAIRS + _C01
            p2 = c2 * HPAIRS + _C2
            for k in range(HPAIRS):
                w01 = plsc.load_gather(t01_v, [p01 + k])
                w2 = plsc.load_gather(t2_v, [p2 + k])
                s = plsc.bitcast(w01, jnp.bfloat16) + plsc.bitcast(w2, jnp.bfloat16)
                lo, hi = plsc.unpack(
                    s, format=plsc.PackFormat.INTERLEAVED,
                    preferred_element_type=jnp.float32,
                )
                out_v[2 * k, r, pl.ds(col, L)] = lo
                out_v[2 * k + 1, r, pl.ds(col, L)] = hi

        pltpu.sync_copy(out_v, out_hbm.at[b, :, pl.ds(brow0, BPR), :])
        return 0

    lax.fori_loop(0, NBLK, block_body, 0)


_rpe_call = functools.partial(
    pl.kernel,
    out_type=jax.ShapeDtypeStruct((B, HEADS, S, S), jnp.float32),
    mesh=plsc.VectorSubcoreMesh(
        core_axis_name="c", subcore_axis_name="s", num_cores=NC, num_subcores=NS
    ),
    scratch_types=[
        pltpu.VMEM((_T01_ROWS * HPAIRS,), jnp.int32),
        pltpu.VMEM((RPE_NUM * HPAIRS,), jnp.int32),
        pltpu.VMEM((3, BPR, S), jnp.int32),
        pltpu.VMEM((HEADS, BPR, S), jnp.float32),
    ],
    compiler_params=pltpu.CompilerParams(
        needs_layout_passes=False, use_tc_tiling_on_sc=True
    ),
)(_rpe_body)


def _pack_pairs(t):
    tb = t.astype(jnp.bfloat16).reshape(t.shape[0], HPAIRS, 2)
    return lax.bitcast_convert_type(tb, jnp.int32).reshape(-1)


def kernel(coord, rpe_table):
    t0 = rpe_table[0:RPE_NUM]
    t1 = rpe_table[RPE_NUM : 2 * RPE_NUM]
    t2 = rpe_table[2 * RPE_NUM : 3 * RPE_NUM]
    t01 = t0[:, None, :] + t1[None, :, :]
    return _rpe_call(
        _pack_pairs(t01.reshape(_T01_ROWS, HEADS)),
        _pack_pairs(t2),
        jnp.transpose(coord, (0, 3, 1, 2)),
    )
